# baseline (speedup 1.0000x reference)
import functools

import jax
import jax.numpy as jnp
from jax import lax
from jax.experimental import pallas as pl
from jax.experimental.pallas import tpu as pltpu
from jax.experimental.pallas import tpu_sc as plsc

NC, NS, LANES = 2, 16, 16
NW = NC * NS
CHUNK = 128
NG = CHUNK // LANES


def _project_linf(W, v):
    m = W.shape[1]
    absW = jnp.abs(W)
    s = jnp.sum(absW, axis=1, keepdims=True)
    u = jnp.sort(absW, axis=1)[:, ::-1]
    css = jnp.cumsum(u, axis=1)
    idx = jnp.arange(1, m + 1)
    cond = u - (css - v) / idx.astype(W.dtype) > 0
    rho = jnp.max(jnp.where(cond, idx, 0), axis=1, keepdims=True)
    theta = (jnp.take_along_axis(css, rho - 1, axis=1) - v) / rho.astype(W.dtype)
    proj = jnp.sign(W) * jnp.maximum(absW - theta, 0.0)
    return jnp.where(s > v, proj, W)


def _mm_body(m_ref, w_ref, o_ref):
    o_ref[...] = jnp.dot(m_ref[...], w_ref[...], preferred_element_type=jnp.float32)


def _tc_matmul(M, Wt, blk):
    n_pad, m = M.shape
    return pl.pallas_call(
        _mm_body,
        grid=(n_pad // blk,),
        in_specs=[
            pl.BlockSpec((blk, m), lambda i: (i, 0)),
            pl.BlockSpec((m, m), lambda i: (0, 0)),
        ],
        out_specs=pl.BlockSpec((blk, m), lambda i: (i, 0)),
        out_shape=jax.ShapeDtypeStruct((n_pad, m), jnp.float32),
    )(M, Wt)


def _sload(ref, i):
    return ref[pl.ds(i, LANES)][0]


def _make_seg_kernel(n_pad, m, e_pad, rows, with_bias):
    mesh = plsc.VectorSubcoreMesh(
        core_axis_name="c", subcore_axis_name="s", num_cores=NC, num_subcores=NS
    )
    acc_words = rows * m

    def body(y_hbm, src_hbm, dst_hbm, w_hbm, meta_hbm, *rest):
        if with_bias:
            (bias_hbm, out_hbm, acc, buf, srcv, dstv, wv, metav, sem) = rest
        else:
            (out_hbm, acc, buf, srcv, dstv, wv, metav, sem) = rest
        c_id = lax.axis_index("c")
        s_id = lax.axis_index("s")
        wid = c_id * NS + s_id
        base_row = wid * rows
        iota = jnp.arange(LANES, dtype=jnp.int32)

        pltpu.sync_copy(meta_hbm, metav)
        start = _sload(metav, wid)
        end = _sload(metav, NW + wid)
        start_al = (start // 8) * 8
        nch = (end - start_al + (CHUNK - 1)) // CHUNK

        if with_bias:
            pltpu.sync_copy(bias_hbm.at[pl.ds(base_row * m, acc_words)], acc)
        else:
            zeros16 = jnp.zeros((LANES,), jnp.float32)

            def zero_body(i, _):
                acc[pl.ds(i * LANES, LANES)] = zeros16
                return 0

            lax.fori_loop(0, acc_words // LANES, zero_body, 0)

        def chunk_body(cc, _):
            off = start_al + cc * CHUNK
            pltpu.sync_copy(src_hbm.at[pl.ds(off, CHUNK)], srcv)
            pltpu.sync_copy(dst_hbm.at[pl.ds(off, CHUNK)], dstv)
            pltpu.sync_copy(w_hbm.at[pl.ds(off, CHUNK)], wv)
            pltpu.async_copy(y_hbm.at[srcv], buf, sem).wait()

            dmuls, w16s, elanes = [], [], []
            for b in range(NG):
                lo = b * LANES
                elane = iota + lo
                eid = off + elane
                valid = (eid >= start) & (eid < end)
                d16 = jnp.where(valid, dstv[pl.ds(lo, LANES)] - base_row, 0)
                w16 = jnp.where(valid, wv[pl.ds(lo, LANES)], 0.0)
                dmuls.append(d16 * m)
                w16s.append(w16)
                elanes.append(elane)

            def f_body(f, _):
                fv = jnp.broadcast_to(f, (LANES,)).astype(jnp.int32)
                for b in range(NG):
                    col = plsc.load_gather(buf, [elanes[b], fv])
                    plsc.addupdate_scatter(acc, [dmuls[b] + fv], col * w16s[b])
                return 0

            lax.fori_loop(0, m, f_body, 0)
            return 0

        lax.fori_loop(0, nch, chunk_body, 0)

        if with_bias:
            def relu_body(i, _):
                v = acc[pl.ds(i * LANES, LANES)]
                acc[pl.ds(i * LANES, LANES)] = jnp.maximum(v, 0.0)
                return 0

            lax.fori_loop(0, acc_words // LANES, relu_body, 0)

        pltpu.sync_copy(acc, out_hbm.at[pl.ds(base_row * m, acc_words)])

    scratch = [
        pltpu.VMEM((acc_words,), jnp.float32),
        pltpu.VMEM((CHUNK, m), jnp.float32),
        pltpu.VMEM((CHUNK,), jnp.int32),
        pltpu.VMEM((CHUNK,), jnp.int32),
        pltpu.VMEM((CHUNK,), jnp.float32),
        pltpu.VMEM((NW * 2 + LANES,), jnp.int32),
        pltpu.SemaphoreType.DMA,
    ]
    return pl.kernel(
        body,
        out_type=jax.ShapeDtypeStruct((n_pad * m,), jnp.float32),
        mesh=mesh,
        scratch_types=scratch,
        compiler_params=pltpu.CompilerParams(use_tc_tiling_on_sc=False, needs_layout_passes=False),
    )


def kernel(X_0, edge_index, edge_weight, U, W, Omega_1, fw_mitr):
    m, n = X_0.shape
    p = U.shape[0]
    E = edge_index.shape[1]
    kappa, A_rho = 0.99, 1.0

    rows = ((n + NW - 1) // NW + 7) // 8 * 8
    n_pad = NW * rows
    e_pad = ((E + 2 * CHUNK + CHUNK - 1) // CHUNK) * CHUNK

    W_p = _project_linf(W, kappa / A_rho)
    Wt = W_p.T

    src = edge_index[0].astype(jnp.int32)
    dst = edge_index[1].astype(jnp.int32)
    order = jnp.argsort(dst)
    src_s = jnp.concatenate([src[order], jnp.zeros((e_pad - E,), jnp.int32)])
    dst_s = jnp.concatenate([dst[order], jnp.zeros((e_pad - E,), jnp.int32)])
    w_s = jnp.concatenate(
        [edge_weight[order].astype(jnp.float32), jnp.zeros((e_pad - E,), jnp.float32)]
    )
    bounds = jnp.searchsorted(
        dst_s[:E], jnp.arange(NW + 1, dtype=jnp.int32) * rows
    ).astype(jnp.int32)
    meta = jnp.concatenate([bounds[:NW], bounds[1 : NW + 1], jnp.zeros((LANES,), jnp.int32)])

    seg_plain = _make_seg_kernel(n_pad, m, e_pad, rows, with_bias=False)
    seg_bias_relu = _make_seg_kernel(n_pad, m, e_pad, rows, with_bias=True)

    ut_pad = jnp.zeros((n_pad, p), jnp.float32).at[:n].set(U.T)
    s1_nm = _tc_matmul(ut_pad, Omega_1.T, blk=1024)
    b_nm = seg_plain(s1_nm, src_s, dst_s, w_s, meta)

    def body(_, M_flat):
        Y = _tc_matmul(M_flat.reshape(n_pad, m), Wt, blk=1024)
        return seg_bias_relu(Y, src_s, dst_s, w_s, meta, b_nm)

    M0 = jnp.zeros((n_pad * m,), jnp.float32)
    M_fin = lax.fori_loop(0, fw_mitr, body, M0)
    return M_fin.reshape(n_pad, m)[:n].T


# speedup vs baseline: 1.3356x; 1.3356x over previous
import functools

import jax
import jax.numpy as jnp
from jax import lax
from jax.experimental import pallas as pl
from jax.experimental.pallas import tpu as pltpu
from jax.experimental.pallas import tpu_sc as plsc

NC, NS, LANES = 2, 16, 16
NW = NC * NS
CHUNK = 128
NG = CHUNK // LANES


def _project_linf(W, v):
    m = W.shape[1]
    absW = jnp.abs(W)
    s = jnp.sum(absW, axis=1, keepdims=True)
    u = jnp.sort(absW, axis=1)[:, ::-1]
    css = jnp.cumsum(u, axis=1)
    idx = jnp.arange(1, m + 1)
    cond = u - (css - v) / idx.astype(W.dtype) > 0
    rho = jnp.max(jnp.where(cond, idx, 0), axis=1, keepdims=True)
    theta = (jnp.take_along_axis(css, rho - 1, axis=1) - v) / rho.astype(W.dtype)
    proj = jnp.sign(W) * jnp.maximum(absW - theta, 0.0)
    return jnp.where(s > v, proj, W)


def _mm_body(m_ref, w_ref, o_ref):
    o_ref[...] = jnp.dot(m_ref[...], w_ref[...], preferred_element_type=jnp.float32)


def _tc_matmul(M, Wt, blk):
    n_pad, m = M.shape
    return pl.pallas_call(
        _mm_body,
        grid=(n_pad // blk,),
        in_specs=[
            pl.BlockSpec((blk, m), lambda i: (i, 0)),
            pl.BlockSpec((m, m), lambda i: (0, 0)),
        ],
        out_specs=pl.BlockSpec((blk, m), lambda i: (i, 0)),
        out_shape=jax.ShapeDtypeStruct((n_pad, m), jnp.float32),
    )(M, Wt)


def _sload(ref, i):
    return ref[pl.ds(i, LANES)][0]


def _make_seg_kernel(n_pad, m, e_pad, rows, with_bias):
    mesh = plsc.VectorSubcoreMesh(
        core_axis_name="c", subcore_axis_name="s", num_cores=NC, num_subcores=NS
    )
    acc_words = rows * m

    def body(y_hbm, src_hbm, dst_hbm, w_hbm, meta_hbm, *rest):
        if with_bias:
            (bias_hbm, out_hbm, acc, buf, srcv, dstv, wv, metav, sem) = rest
        else:
            (out_hbm, acc, buf, srcv, dstv, wv, metav, sem) = rest
        c_id = lax.axis_index("c")
        s_id = lax.axis_index("s")
        wid = c_id * NS + s_id
        base_row = wid * rows
        iota = jnp.arange(LANES, dtype=jnp.int32)

        pltpu.sync_copy(meta_hbm, metav)
        start = _sload(metav, wid)
        end = _sload(metav, NW + wid)
        start_al = (start // 8) * 8
        nch = (end - start_al + (CHUNK - 1)) // CHUNK

        if with_bias:
            pltpu.sync_copy(bias_hbm.at[pl.ds(base_row * m, acc_words)], acc)
        else:
            zeros16 = jnp.zeros((LANES,), jnp.float32)

            def zero_body(i, _):
                acc[pl.ds(i * LANES, LANES)] = zeros16
                return 0

            lax.fori_loop(0, acc_words // LANES, zero_body, 0)

        def chunk_body(cc, _):
            off = start_al + cc * CHUNK
            pltpu.sync_copy(src_hbm.at[pl.ds(off, CHUNK)], srcv)
            pltpu.sync_copy(dst_hbm.at[pl.ds(off, CHUNK)], dstv)
            pltpu.sync_copy(w_hbm.at[pl.ds(off, CHUNK)], wv)
            pltpu.async_copy(y_hbm.at[srcv], buf, sem).wait()

            dmuls, w16s, elanes = [], [], []
            for b in range(NG):
                lo = b * LANES
                elane = iota + lo
                eid = off + elane
                valid = (eid >= start) & (eid < end)
                d16 = jnp.where(valid, dstv[pl.ds(lo, LANES)] - base_row, 0)
                w16 = jnp.where(valid, wv[pl.ds(lo, LANES)], 0.0)
                dmuls.append(d16 * m)
                w16s.append(w16)
                elanes.append(elane)

            def f_body(f):
                fv = jnp.broadcast_to(f, (LANES,)).astype(jnp.int32)
                for b in range(NG):
                    col = plsc.load_gather(buf, [elanes[b], fv])
                    plsc.addupdate_scatter(acc, [dmuls[b] + fv], col * w16s[b])

            plsc.parallel_loop(0, m, 1, unroll=8)(f_body)
            return 0

        lax.fori_loop(0, nch, chunk_body, 0)

        if with_bias:
            def relu_body(i, _):
                v = acc[pl.ds(i * LANES, LANES)]
                acc[pl.ds(i * LANES, LANES)] = jnp.maximum(v, 0.0)
                return 0

            lax.fori_loop(0, acc_words // LANES, relu_body, 0)

        pltpu.sync_copy(acc, out_hbm.at[pl.ds(base_row * m, acc_words)])

    scratch = [
        pltpu.VMEM((acc_words,), jnp.float32),
        pltpu.VMEM((CHUNK, m), jnp.float32),
        pltpu.VMEM((CHUNK,), jnp.int32),
        pltpu.VMEM((CHUNK,), jnp.int32),
        pltpu.VMEM((CHUNK,), jnp.float32),
        pltpu.VMEM((NW * 2 + LANES,), jnp.int32),
        pltpu.SemaphoreType.DMA,
    ]
    return pl.kernel(
        body,
        out_type=jax.ShapeDtypeStruct((n_pad * m,), jnp.float32),
        mesh=mesh,
        scratch_types=scratch,
        compiler_params=pltpu.CompilerParams(use_tc_tiling_on_sc=False, needs_layout_passes=False),
    )


def kernel(X_0, edge_index, edge_weight, U, W, Omega_1, fw_mitr):
    m, n = X_0.shape
    p = U.shape[0]
    E = edge_index.shape[1]
    kappa, A_rho = 0.99, 1.0

    rows = ((n + NW - 1) // NW + 7) // 8 * 8
    n_pad = NW * rows
    e_pad = ((E + 2 * CHUNK + CHUNK - 1) // CHUNK) * CHUNK

    W_p = _project_linf(W, kappa / A_rho)
    Wt = W_p.T

    src = edge_index[0].astype(jnp.int32)
    dst = edge_index[1].astype(jnp.int32)
    order = jnp.argsort(dst)
    src_s = jnp.concatenate([src[order], jnp.zeros((e_pad - E,), jnp.int32)])
    dst_s = jnp.concatenate([dst[order], jnp.zeros((e_pad - E,), jnp.int32)])
    w_s = jnp.concatenate(
        [edge_weight[order].astype(jnp.float32), jnp.zeros((e_pad - E,), jnp.float32)]
    )
    bounds = jnp.searchsorted(
        dst_s[:E], jnp.arange(NW + 1, dtype=jnp.int32) * rows
    ).astype(jnp.int32)
    meta = jnp.concatenate([bounds[:NW], bounds[1 : NW + 1], jnp.zeros((LANES,), jnp.int32)])

    seg_plain = _make_seg_kernel(n_pad, m, e_pad, rows, with_bias=False)
    seg_bias_relu = _make_seg_kernel(n_pad, m, e_pad, rows, with_bias=True)

    ut_pad = jnp.zeros((n_pad, p), jnp.float32).at[:n].set(U.T)
    s1_nm = _tc_matmul(ut_pad, Omega_1.T, blk=1024)
    b_nm = seg_plain(s1_nm, src_s, dst_s, w_s, meta)

    def body(_, M_flat):
        Y = _tc_matmul(M_flat.reshape(n_pad, m), Wt, blk=1024)
        return seg_bias_relu(Y, src_s, dst_s, w_s, meta, b_nm)

    M0 = jnp.zeros((n_pad * m,), jnp.float32)
    M_fin = lax.fori_loop(0, fw_mitr, body, M0)
    return M_fin.reshape(n_pad, m)[:n].T


# speedup vs baseline: 8.3223x; 6.2311x over previous
import functools

import jax
import jax.numpy as jnp
from jax import lax
from jax.experimental import pallas as pl
from jax.experimental.pallas import tpu as pltpu
from jax.experimental.pallas import tpu_sc as plsc

NC, NS, LANES = 2, 16, 16
NW = NC * NS
CHUNK = 128
NG = CHUNK // LANES


def _project_linf(W, v):
    m = W.shape[1]
    absW = jnp.abs(W)
    s = jnp.sum(absW, axis=1, keepdims=True)
    u = jnp.sort(absW, axis=1)[:, ::-1]
    css = jnp.cumsum(u, axis=1)
    idx = jnp.arange(1, m + 1)
    cond = u - (css - v) / idx.astype(W.dtype) > 0
    rho = jnp.max(jnp.where(cond, idx, 0), axis=1, keepdims=True)
    theta = (jnp.take_along_axis(css, rho - 1, axis=1) - v) / rho.astype(W.dtype)
    proj = jnp.sign(W) * jnp.maximum(absW - theta, 0.0)
    return jnp.where(s > v, proj, W)


def _mm_body(m_ref, w_ref, o_ref):
    o_ref[...] = jnp.dot(m_ref[...], w_ref[...], preferred_element_type=jnp.float32)


def _tc_matmul(M, Wt, blk):
    n_pad, m = M.shape
    return pl.pallas_call(
        _mm_body,
        grid=(n_pad // blk,),
        in_specs=[
            pl.BlockSpec((blk, m), lambda i: (i, 0)),
            pl.BlockSpec((m, m), lambda i: (0, 0)),
        ],
        out_specs=pl.BlockSpec((blk, m), lambda i: (i, 0)),
        out_shape=jax.ShapeDtypeStruct((n_pad, m), jnp.float32),
    )(M, Wt)


def _sload(ref, i):
    return ref[pl.ds(i, LANES)][0]


def _make_seg_kernel(n_pad, m, e_pad, rows, with_bias):
    mesh = plsc.VectorSubcoreMesh(
        core_axis_name="c", subcore_axis_name="s", num_cores=NC, num_subcores=NS
    )
    acc_words = rows * m

    def body(y_hbm, src_hbm, dst_hbm, w_hbm, meta_hbm, *rest):
        if with_bias:
            (bias_hbm, out_hbm, acc, buf, srcv, dstv, wv, metav, sem) = rest
        else:
            (out_hbm, acc, buf, srcv, dstv, wv, metav, sem) = rest
        c_id = lax.axis_index("c")
        s_id = lax.axis_index("s")
        wid = c_id * NS + s_id
        base_row = wid * rows
        iota = jnp.arange(LANES, dtype=jnp.int32)

        pltpu.sync_copy(meta_hbm, metav)
        start = _sload(metav, wid)
        end = _sload(metav, NW + wid)
        start_al = (start // 8) * 8
        nch = (end - start_al + (CHUNK - 1)) // CHUNK

        if with_bias:
            pltpu.sync_copy(bias_hbm.at[pl.ds(base_row * m, acc_words)], acc)
        else:
            zeros16 = jnp.zeros((LANES,), jnp.float32)

            def zero_body(i, _):
                acc[pl.ds(i * LANES, LANES)] = zeros16
                return 0

            lax.fori_loop(0, acc_words // LANES, zero_body, 0)

        def chunk_body(cc, _):
            off = start_al + cc * CHUNK
            pltpu.sync_copy(src_hbm.at[pl.ds(off, CHUNK)], srcv)
            pltpu.sync_copy(dst_hbm.at[pl.ds(off, CHUNK)], dstv.at[pl.ds(0, CHUNK)])
            pltpu.sync_copy(w_hbm.at[pl.ds(off, CHUNK)], wv.at[pl.ds(0, CHUNK)])
            pltpu.async_copy(y_hbm.at[srcv], buf, sem).wait()

            def edge_body(e):
                eid = off + e
                valid = (eid >= start) & (eid < end)
                d = jnp.where(valid, _sload(dstv, e) - base_row, 0)
                w = jnp.where(valid, _sload(wv, e), 0.0)
                wvec = jnp.broadcast_to(w, (LANES,))
                abase = d * m
                for j in range(m // LANES):
                    x = buf[e, pl.ds(j * LANES, LANES)]
                    plsc.addupdate(acc.at[pl.ds(abase + j * LANES, LANES)], x * wvec)

            plsc.parallel_loop(0, CHUNK, 1, unroll=2)(edge_body)
            return 0

        lax.fori_loop(0, nch, chunk_body, 0)

        if with_bias:
            def relu_body(i, _):
                v = acc[pl.ds(i * LANES, LANES)]
                acc[pl.ds(i * LANES, LANES)] = jnp.maximum(v, 0.0)
                return 0

            lax.fori_loop(0, acc_words // LANES, relu_body, 0)

        pltpu.sync_copy(acc, out_hbm.at[pl.ds(base_row * m, acc_words)])

    scratch = [
        pltpu.VMEM((acc_words,), jnp.float32),
        pltpu.VMEM((CHUNK, m), jnp.float32),
        pltpu.VMEM((CHUNK,), jnp.int32),
        pltpu.VMEM((CHUNK + LANES,), jnp.int32),
        pltpu.VMEM((CHUNK + LANES,), jnp.float32),
        pltpu.VMEM((NW * 2 + LANES,), jnp.int32),
        pltpu.SemaphoreType.DMA,
    ]
    return pl.kernel(
        body,
        out_type=jax.ShapeDtypeStruct((n_pad * m,), jnp.float32),
        mesh=mesh,
        scratch_types=scratch,
        compiler_params=pltpu.CompilerParams(use_tc_tiling_on_sc=False, needs_layout_passes=False),
    )


def kernel(X_0, edge_index, edge_weight, U, W, Omega_1, fw_mitr):
    m, n = X_0.shape
    p = U.shape[0]
    E = edge_index.shape[1]
    kappa, A_rho = 0.99, 1.0

    rows = ((n + NW - 1) // NW + 7) // 8 * 8
    n_pad = NW * rows
    e_pad = ((E + 2 * CHUNK + CHUNK - 1) // CHUNK) * CHUNK

    W_p = _project_linf(W, kappa / A_rho)
    Wt = W_p.T

    src = edge_index[0].astype(jnp.int32)
    dst = edge_index[1].astype(jnp.int32)
    order = jnp.argsort(dst)
    src_s = jnp.concatenate([src[order], jnp.zeros((e_pad - E,), jnp.int32)])
    dst_s = jnp.concatenate([dst[order], jnp.zeros((e_pad - E,), jnp.int32)])
    w_s = jnp.concatenate(
        [edge_weight[order].astype(jnp.float32), jnp.zeros((e_pad - E,), jnp.float32)]
    )
    bounds = jnp.searchsorted(
        dst_s[:E], jnp.arange(NW + 1, dtype=jnp.int32) * rows
    ).astype(jnp.int32)
    meta = jnp.concatenate([bounds[:NW], bounds[1 : NW + 1], jnp.zeros((LANES,), jnp.int32)])

    seg_plain = _make_seg_kernel(n_pad, m, e_pad, rows, with_bias=False)
    seg_bias_relu = _make_seg_kernel(n_pad, m, e_pad, rows, with_bias=True)

    ut_pad = jnp.zeros((n_pad, p), jnp.float32).at[:n].set(U.T)
    s1_nm = _tc_matmul(ut_pad, Omega_1.T, blk=1024)
    b_nm = seg_plain(s1_nm, src_s, dst_s, w_s, meta)

    def body(_, M_flat):
        Y = _tc_matmul(M_flat.reshape(n_pad, m), Wt, blk=1024)
        return seg_bias_relu(Y, src_s, dst_s, w_s, meta, b_nm)

    M0 = jnp.zeros((n_pad * m,), jnp.float32)
    M_fin = lax.fori_loop(0, fw_mitr, body, M0)
    return M_fin.reshape(n_pad, m)[:n].T
